# baseline (speedup 1.0000x reference)
import functools

import jax
import jax.numpy as jnp
from jax import lax
from jax.experimental import pallas as pl
from jax.experimental.pallas import tpu as pltpu
from jax.experimental.pallas import tpu_sc as plsc

CP = 16
WPAD = 128
NC = 2
NS = 16
NW = NC * NS
CHUNK = 128
L = 16


def _mm_body(x_ref, w_ref, b_ref, o_ref):
    c = w_ref.shape[0]
    y = lax.dot_general(x_ref[...], w_ref[...], (((1,), (1,)), ((), ())),
                        preferred_element_type=jnp.float32)
    y = y + b_ref[...] * (1.0 / 3.0)
    o_ref[...] = jnp.pad(y, ((0, 0), (0, WPAD - c)))


def _project(x, w, b1):
    bs, h = x.shape
    c = w.shape[0]
    blk = 1024
    while bs % blk != 0:
        blk //= 2
    return pl.pallas_call(
        _mm_body,
        grid=(bs // blk,),
        in_specs=[
            pl.BlockSpec((blk, h), lambda i: (i, 0)),
            pl.BlockSpec((c, h), lambda i: (0, 0)),
            pl.BlockSpec((1, c), lambda i: (0, 0)),
        ],
        out_specs=pl.BlockSpec((blk, WPAD), lambda i: (i, 0)),
        out_shape=jax.ShapeDtypeStruct((bs, WPAD), jnp.float32),
        compiler_params=pltpu.CompilerParams(
            vmem_limit_bytes=100 * 1024 * 1024),
    )(x, w, b1)


def _make_sc_gather(B, T, S, C):
    wpb = NW // B
    trip_w = (B * T) // NW
    n_ids = 3 * trip_w
    n_blk = n_ids // CHUNK
    mesh = plsc.VectorSubcoreMesh(core_axis_name="c", subcore_axis_name="s")

    @functools.partial(
        pl.kernel,
        mesh=mesh,
        out_type=jax.ShapeDtypeStruct((C, T // CHUNK, B, CHUNK), jnp.float32),
        scratch_types=[
            pltpu.VMEM((n_ids,), jnp.int32),
            pltpu.VMEM((n_ids, CP), jnp.float32),
            pltpu.VMEM((C, trip_w), jnp.float32),
            pltpu.SemaphoreType.DMA,
        ],
        compiler_params=pltpu.CompilerParams(use_tc_tiling_on_sc=False,
                                             needs_layout_passes=False),
    )
    def sc_kernel(table_hbm, idx_hbm, out_hbm, idx_v, rows_v, outT_v, sem):
        wid = lax.axis_index("s") * NC + lax.axis_index("c")
        bb = wid // wpb
        woff = wid % wpb
        tpw = trip_w // CHUNK
        for a in range(3):
            for k in range(tpw):
                pltpu.sync_copy(
                    idx_hbm.at[a, woff * tpw + k, bb],
                    idx_v.at[pl.ds(a * trip_w + k * CHUNK, CHUNK)])
        base = jnp.full((L,), bb * S, jnp.int32)
        mul8 = jnp.full((L,), WPAD // CP, jnp.int32)

        def add_base(i, _):
            idx_v[pl.ds(i * L, L)] = (idx_v[pl.ds(i * L, L)] + base) * mul8
            return 0

        lax.fori_loop(0, n_ids // L, add_base, 0)
        copies = []
        for j in range(n_blk):
            copies.append(pltpu.async_copy(
                table_hbm.at[idx_v.at[pl.ds(j * CHUNK, CHUNK)]],
                rows_v.at[pl.ds(j * CHUNK, CHUNK)], sem))
        for cp in copies:
            cp.wait()
        lane = lax.iota(jnp.int32, L)

        for c in range(C):
            cc = jnp.full((L,), c, jnp.int32)

            def body(j, _, cc=cc, c=c):
                r = j * L + lane
                v = (plsc.load_gather(rows_v, [r, cc])
                     + plsc.load_gather(rows_v, [r + trip_w, cc])
                     + plsc.load_gather(rows_v, [r + 2 * trip_w, cc]))
                outT_v[c, pl.ds(j * L, L)] = v
                return 0

            lax.fori_loop(0, trip_w // L, body, 0)
        for c in range(C):
            for k in range(trip_w // CHUNK):
                pltpu.sync_copy(
                    outT_v.at[c, pl.ds(k * CHUNK, CHUNK)],
                    out_hbm.at[c, woff * (trip_w // CHUNK) + k, bb])

    return sc_kernel


def kernel(hidden_states, triple_anchor_ids, W, b):
    B, S, H = hidden_states.shape
    _, T, _ = triple_anchor_ids.shape
    C = W.shape[0]
    BS = B * S
    N = B * T

    assert NW % B == 0 and (3 * N) % (NW * CHUNK) == 0

    P = _project(hidden_states.reshape(BS, H), W, b.reshape(1, C))
    table = P.reshape(BS * (WPAD // CP), CP)
    idx = (triple_anchor_ids.astype(jnp.int32).transpose(2, 1, 0)
           .reshape(3, T // CHUNK, CHUNK, B).transpose(0, 1, 3, 2))

    out = _make_sc_gather(B, T, S, C)(table, idx)

    return out.transpose(2, 1, 3, 0).reshape(B, T, C)


# speedup vs baseline: 1.0651x; 1.0651x over previous
import functools

import jax
import jax.numpy as jnp
from jax import lax
from jax.experimental import pallas as pl
from jax.experimental.pallas import tpu as pltpu
from jax.experimental.pallas import tpu_sc as plsc

CP = 16
WPAD = 128
NC = 2
NS = 16
NW = NC * NS
CHUNK = 128
L = 16


def _mm_body(x_ref, w_ref, b_ref, o_ref):
    c = w_ref.shape[0]
    y = lax.dot_general(x_ref[...], w_ref[...], (((1,), (1,)), ((), ())),
                        preferred_element_type=jnp.float32)
    y = y + b_ref[...] * (1.0 / 3.0)
    o_ref[...] = jnp.pad(y, ((0, 0), (0, WPAD - c)))


def _project(x, w, b1):
    bs, h = x.shape
    c = w.shape[0]
    blk = 2048
    while bs % blk != 0:
        blk //= 2
    return pl.pallas_call(
        _mm_body,
        grid=(bs // blk,),
        in_specs=[
            pl.BlockSpec((blk, h), lambda i: (i, 0)),
            pl.BlockSpec((c, h), lambda i: (0, 0)),
            pl.BlockSpec((1, c), lambda i: (0, 0)),
        ],
        out_specs=pl.BlockSpec((blk, WPAD), lambda i: (i, 0)),
        out_shape=jax.ShapeDtypeStruct((bs, WPAD), jnp.float32),
        compiler_params=pltpu.CompilerParams(
            vmem_limit_bytes=100 * 1024 * 1024),
    )(x, w, b1)


def _make_sc_gather(B, T, S, C):
    wpb = NW // B
    trip_w = (B * T) // NW
    n_ids = 3 * trip_w
    n_blk = n_ids // CHUNK
    mesh = plsc.VectorSubcoreMesh(core_axis_name="c", subcore_axis_name="s")

    @functools.partial(
        pl.kernel,
        mesh=mesh,
        out_type=jax.ShapeDtypeStruct((C, T // CHUNK, B, CHUNK), jnp.float32),
        scratch_types=[
            pltpu.VMEM((n_ids,), jnp.int32),
            pltpu.VMEM((n_ids, CP), jnp.float32),
            pltpu.VMEM((C, trip_w), jnp.float32),
            pltpu.SemaphoreType.DMA,
        ],
        compiler_params=pltpu.CompilerParams(use_tc_tiling_on_sc=False,
                                             needs_layout_passes=False),
    )
    def sc_kernel(table_hbm, idx_hbm, out_hbm, idx_v, rows_v, outT_v, sem):
        wid = lax.axis_index("s") * NC + lax.axis_index("c")
        bb = wid // wpb
        woff = wid % wpb
        tpw = trip_w // CHUNK
        for a in range(3):
            for k in range(tpw):
                pltpu.sync_copy(
                    idx_hbm.at[a, woff * tpw + k, bb],
                    idx_v.at[pl.ds(a * trip_w + k * CHUNK, CHUNK)])
        base = jnp.full((L,), bb * S, jnp.int32)
        mul8 = jnp.full((L,), WPAD // CP, jnp.int32)

        def add_base(i, _):
            idx_v[pl.ds(i * L, L)] = (idx_v[pl.ds(i * L, L)] + base) * mul8
            return 0

        lax.fori_loop(0, n_ids // L, add_base, 0)
        copies = []
        for j in range(n_blk):
            copies.append(pltpu.async_copy(
                table_hbm.at[idx_v.at[pl.ds(j * CHUNK, CHUNK)]],
                rows_v.at[pl.ds(j * CHUNK, CHUNK)], sem))
        for cp in copies:
            cp.wait()
        lane = lax.iota(jnp.int32, L)

        for c in range(C):
            cc = jnp.full((L,), c, jnp.int32)

            def body(j, _, cc=cc, c=c):
                r = j * L + lane
                v = (plsc.load_gather(rows_v, [r, cc])
                     + plsc.load_gather(rows_v, [r + trip_w, cc])
                     + plsc.load_gather(rows_v, [r + 2 * trip_w, cc]))
                outT_v[c, pl.ds(j * L, L)] = v
                return 0

            lax.fori_loop(0, trip_w // L, body, 0)
        for c in range(C):
            for k in range(trip_w // CHUNK):
                pltpu.sync_copy(
                    outT_v.at[c, pl.ds(k * CHUNK, CHUNK)],
                    out_hbm.at[c, woff * (trip_w // CHUNK) + k, bb])

    return sc_kernel


def kernel(hidden_states, triple_anchor_ids, W, b):
    B, S, H = hidden_states.shape
    _, T, _ = triple_anchor_ids.shape
    C = W.shape[0]
    BS = B * S
    N = B * T

    assert NW % B == 0 and (3 * N) % (NW * CHUNK) == 0

    P = _project(hidden_states.reshape(BS, H), W, b.reshape(1, C))
    table = P.reshape(BS * (WPAD // CP), CP)
    idx = (triple_anchor_ids.astype(jnp.int32).transpose(2, 1, 0)
           .reshape(3, T // CHUNK, CHUNK, B).transpose(0, 1, 3, 2))

    out = _make_sc_gather(B, T, S, C)(table, idx)

    return out.transpose(2, 1, 3, 0).reshape(B, T, C)
